# Initial kernel scaffold; baseline (speedup 1.0000x reference)
import jax
import jax.numpy as jnp
from jax.experimental import pallas as pl


def kernel(feat, adj_loop, diag_mat, W, b):
    raise NotImplementedError("write your pallas kernel here")


# speedup vs baseline: 1.4913x; 1.4913x over previous
import functools

import jax
import jax.numpy as jnp
from jax.experimental import pallas as pl


def _presup_kernel(feat_ref, w_ref, b_ref, pcat_ref):
    p = jnp.dot(feat_ref[...], w_ref[...].T, preferred_element_type=jnp.float32)
    p = p + b_ref[...]
    pcat_ref[...] = jnp.concatenate([p, p * p], axis=1)


def _bilinear_kernel(adj_ref, pcat_ref, x_ref, *, d):
    sq = jnp.dot(adj_ref[...], pcat_ref[...], preferred_element_type=jnp.float32)
    s = sq[:, :d]
    q = sq[:, d:]
    x_ref[...] = 0.5 * (s * s - q)


def _rowmm_kernel(diag_ref, x_ref, out_ref):
    out_ref[...] = jnp.dot(diag_ref[...], x_ref[...], preferred_element_type=jnp.float32)


def kernel(feat, adj_loop, diag_mat, W, b):
    n, _ = feat.shape
    d = W.shape[0]
    bm = 400 if n % 400 == 0 else n

    pcat = pl.pallas_call(
        _presup_kernel,
        out_shape=jax.ShapeDtypeStruct((n, 2 * d), jnp.float32),
    )(feat, W, b.reshape(1, d))

    x = pl.pallas_call(
        functools.partial(_bilinear_kernel, d=d),
        grid=(n // bm,),
        in_specs=[
            pl.BlockSpec((bm, n), lambda i: (i, 0)),
            pl.BlockSpec((n, 2 * d), lambda i: (0, 0)),
        ],
        out_specs=pl.BlockSpec((bm, d), lambda i: (i, 0)),
        out_shape=jax.ShapeDtypeStruct((n, d), jnp.float32),
    )(adj_loop, pcat)

    out = pl.pallas_call(
        _rowmm_kernel,
        grid=(n // bm,),
        in_specs=[
            pl.BlockSpec((bm, n), lambda i: (i, 0)),
            pl.BlockSpec((n, d), lambda i: (0, 0)),
        ],
        out_specs=pl.BlockSpec((bm, d), lambda i: (i, 0)),
        out_shape=jax.ShapeDtypeStruct((n, d), jnp.float32),
    )(diag_mat, x)
    return out


# speedup vs baseline: 1.5419x; 1.0339x over previous
import functools

import jax
import jax.numpy as jnp
from jax.experimental import pallas as pl
from jax.experimental.pallas import tpu as pltpu


def _fused_kernel(feat_ref, w_ref, b_ref, adj_ref, diag_ref, out_ref,
                  pcat_ref, x_ref, *, g, bm, d):
    i = pl.program_id(0)

    @pl.when(i == 0)
    def _init():
        p = jnp.dot(feat_ref[...], w_ref[...].T,
                    preferred_element_type=jnp.float32) + b_ref[...]
        pcat_ref[:, :d] = p
        pcat_ref[:, d:] = p * p

    @pl.when(i < g)
    def _phase_adj():
        sq = jnp.dot(adj_ref[...], pcat_ref[...],
                     preferred_element_type=jnp.float32)
        s = sq[:, :d]
        q = sq[:, d:]
        x_ref[pl.ds(i * bm, bm), :] = 0.5 * (s * s - q)

    @pl.when(i >= g)
    def _phase_diag():
        out_ref[...] = jnp.dot(diag_ref[...], x_ref[...],
                               preferred_element_type=jnp.float32)


def kernel(feat, adj_loop, diag_mat, W, b):
    n, _ = feat.shape
    d = W.shape[0]
    bm = 200 if n % 200 == 0 else n
    g = n // bm

    return pl.pallas_call(
        functools.partial(_fused_kernel, g=g, bm=bm, d=d),
        grid=(2 * g,),
        in_specs=[
            pl.BlockSpec((n, feat.shape[1]), lambda i: (0, 0)),
            pl.BlockSpec((d, W.shape[1]), lambda i: (0, 0)),
            pl.BlockSpec((1, d), lambda i: (0, 0)),
            pl.BlockSpec((bm, n), lambda i: (jnp.minimum(i, g - 1), 0)),
            pl.BlockSpec((bm, n), lambda i: (jnp.maximum(i - g, 0), 0)),
        ],
        out_specs=pl.BlockSpec((bm, d), lambda i: (jnp.maximum(i - g, 0), 0)),
        out_shape=jax.ShapeDtypeStruct((n, d), jnp.float32),
        scratch_shapes=[
            pltpu.VMEM((n, 2 * d), jnp.float32),
            pltpu.VMEM((n, d), jnp.float32),
        ],
    )(feat, W, b.reshape(1, d), adj_loop, diag_mat)
